# Initial kernel scaffold; baseline (speedup 1.0000x reference)
import jax
import jax.numpy as jnp
from jax.experimental import pallas as pl


def kernel(x, edge_index, edge_attr, op, W1, b1, W2, b2, Wp, bp, Wfc, bfc):
    raise NotImplementedError("write your pallas kernel here")


# speedup vs baseline: 4.5029x; 4.5029x over previous
import functools

import jax
import jax.numpy as jnp
from jax import lax
from jax.experimental import pallas as pl
from jax.experimental.pallas import tpu as pltpu
from jax.experimental.pallas import tpu_sc as plsc

_N = 10000
_E = 320000
_H = 64
_CHUNK = 128
_NCHUNKS = _E // _CHUNK
_NC = 2
_NS = 16
_NW = _NC * _NS
_BASE = _NCHUNKS // _NW
_REM = _NCHUNKS % _NW
_RPS = _N // _NS


def _mesh():
    return plsc.VectorSubcoreMesh(core_axis_name="c", subcore_axis_name="s")


def _worker_id():
    return lax.axis_index("s") * _NC + lax.axis_index("c")


def _chunk_range(w):
    start = w * _BASE + jnp.minimum(w, _REM)
    count = _BASE + (w < _REM).astype(jnp.int32)
    return start, start + count


def _sc_deg_body(attr_hbm, dst_hbm, ew_hbm, deg_hbm, dst_v, attr_v, ew_v,
                 deg_local):
    w = _worker_id()

    @pl.loop(0, _N // 16)
    def _zero(i):
        deg_local[pl.ds(i * 16, 16)] = jnp.zeros((16,), jnp.float32)

    lo, hi = _chunk_range(w)

    @pl.loop(lo, hi)
    def _chunk(g):
        off = g * _CHUNK
        pltpu.sync_copy(dst_hbm.at[pl.ds(off, _CHUNK)], dst_v)
        pltpu.sync_copy(attr_hbm.at[pl.ds(off, _CHUNK)], attr_v)

        @pl.loop(0, _CHUNK // 16)
        def _grp(j):
            d16 = dst_v[pl.ds(j * 16, 16)]
            a16 = attr_v[pl.ds(j * 16, 16)]
            e16 = 1.0 / (a16 + 1.0)
            ew_v[pl.ds(j * 16, 16)] = e16
            plsc.addupdate_scatter(deg_local, [d16], e16)

        pltpu.sync_copy(ew_v, ew_hbm.at[pl.ds(off, _CHUNK)])

    pltpu.sync_copy(deg_local, deg_hbm.at[w])


def _sc_deg(attr, dst):
    kern = functools.partial(
        pl.kernel,
        compiler_params=pltpu.CompilerParams(needs_layout_passes=False, use_tc_tiling_on_sc=False),
        out_type=(
            jax.ShapeDtypeStruct((_E,), jnp.float32),
            jax.ShapeDtypeStruct((_NW, _N), jnp.float32),
        ),
        mesh=_mesh(),
        scratch_types=[
            pltpu.VMEM((_CHUNK,), jnp.int32),
            pltpu.VMEM((_CHUNK,), jnp.float32),
            pltpu.VMEM((_CHUNK,), jnp.float32),
            pltpu.VMEM((_N,), jnp.float32),
        ],
    )(_sc_deg_body)
    return kern(attr, dst)


def _sc_msg_body(hp_hbm, src_hbm, dst_hbm, ew_hbm, out_hbm, src_v, dst_v,
                 ew_v, rows_v, zbuf, acc_sh, sem):
    c = lax.axis_index("c")
    s = lax.axis_index("s")
    w = s * _NC + c

    @pl.loop(0, 125)
    def _zrow(i):
        for j in range(_H // 16):
            zbuf[i, pl.ds(j * 16, 16)] = jnp.zeros((16,), jnp.float32)

    for k in range(_RPS // 125):
        pltpu.sync_copy(zbuf, acc_sh.at[pl.ds(s * _RPS + k * 125, 125)])
    plsc.subcore_barrier()

    lo, hi = _chunk_range(w)

    @pl.loop(lo, hi)
    def _chunk(g):
        off = g * _CHUNK
        pltpu.sync_copy(src_hbm.at[pl.ds(off, _CHUNK)], src_v)
        pltpu.sync_copy(dst_hbm.at[pl.ds(off, _CHUNK)], dst_v)
        pltpu.sync_copy(ew_hbm.at[pl.ds(off, _CHUNK)], ew_v)
        pltpu.async_copy(hp_hbm.at[src_v], rows_v, sem).wait()

        @pl.loop(0, _CHUNK // 16)
        def _grp(j):
            ew16 = ew_v[pl.ds(j * 16, 16)]
            ridx = lax.broadcasted_iota(jnp.int32, (16,), 0) + j * 16
            for f in range(_H):
                cidx = jnp.full((16,), f, jnp.int32)
                col = plsc.load_gather(rows_v, [ridx, cidx])
                plsc.store_scatter(rows_v, [ridx, cidx], col * ew16)

        pltpu.sync_copy(rows_v, acc_sh.at[dst_v], add=True)

    plsc.subcore_barrier()
    pltpu.sync_copy(acc_sh.at[pl.ds(s * _RPS, _RPS)],
                    out_hbm.at[c, pl.ds(s * _RPS, _RPS)])


def _sc_msg(hp, src, dst, ew):
    kern = functools.partial(
        pl.kernel,
        compiler_params=pltpu.CompilerParams(needs_layout_passes=False, use_tc_tiling_on_sc=False),
        out_type=jax.ShapeDtypeStruct((_NC, _N, _H), jnp.float32),
        mesh=_mesh(),
        scratch_types=[
            pltpu.VMEM((_CHUNK,), jnp.int32),
            pltpu.VMEM((_CHUNK,), jnp.int32),
            pltpu.VMEM((_CHUNK,), jnp.float32),
            pltpu.VMEM((_CHUNK, _H), jnp.float32),
            pltpu.VMEM((125, _H), jnp.float32),
            pltpu.VMEM_SHARED((_N, _H), jnp.float32),
            pltpu.SemaphoreType.DMA,
        ],
    )(_sc_msg_body)
    return kern(hp, src, dst, ew)


def _tc1_body(x_ref, w1_ref, degp_ref, h1_ref, hp1_ref, dinv_ref):
    deg = jnp.sum(degp_ref[...], axis=0)[:, None] + 1.0
    dinv = jnp.where(deg > 0, lax.rsqrt(jnp.maximum(deg, 1e-12)), 0.0)
    h1 = jnp.dot(x_ref[...], w1_ref[...], preferred_element_type=jnp.float32)
    h1_ref[...] = h1
    hp1_ref[...] = h1 * dinv
    dinv_ref[...] = dinv


def _tc1(x, W1, deg_parts):
    return pl.pallas_call(
        _tc1_body,
        out_shape=(
            jax.ShapeDtypeStruct((_N, _H), jnp.float32),
            jax.ShapeDtypeStruct((_N, _H), jnp.float32),
            jax.ShapeDtypeStruct((_N, 1), jnp.float32),
        ),
    )(x, W1, deg_parts)


def _tc2_body(agg_ref, h1_ref, dinv_ref, w2_ref, b1_ref, h2_ref, hp2_ref):
    dinv = dinv_ref[...]
    a = agg_ref[...]
    z = dinv * (a[0] + a[1]) + (dinv * dinv) * h1_ref[...] + b1_ref[...]
    r = jnp.maximum(z, 0.0)
    h2 = jnp.dot(r, w2_ref[...], preferred_element_type=jnp.float32)
    h2_ref[...] = h2
    hp2_ref[...] = h2 * dinv


def _tc2(agg1, h1, dinv, W2, b1):
    return pl.pallas_call(
        _tc2_body,
        out_shape=(
            jax.ShapeDtypeStruct((_N, _H), jnp.float32),
            jax.ShapeDtypeStruct((_N, _H), jnp.float32),
        ),
    )(agg1, h1, dinv, W2, b1)


def _tc3_body(agg_ref, h2_ref, dinv_ref, op_ref, wp_ref, bp_ref, wfc_ref,
              bfc_ref, b2_ref, out_ref):
    dinv = dinv_ref[...]
    a = agg_ref[...]
    z = dinv * (a[0] + a[1]) + (dinv * dinv) * h2_ref[...] + b2_ref[...]
    r = jnp.maximum(z, 0.0)
    emb = jnp.dot(r, wp_ref[...], preferred_element_type=jnp.float32) \
        + bp_ref[...]
    wfc = wfc_ref[...]
    out = jnp.dot(emb, wfc[:128], preferred_element_type=jnp.float32) \
        + jnp.dot(op_ref[...], wfc[128:], preferred_element_type=jnp.float32) \
        + bfc_ref[...]
    out_ref[...] = out


def _tc3(agg2, h2, dinv, op, Wp, bp, Wfc, bfc, b2):
    return pl.pallas_call(
        _tc3_body,
        out_shape=jax.ShapeDtypeStruct((_N, 1), jnp.float32),
    )(agg2, h2, dinv, op, Wp, bp, Wfc, bfc, b2)


def kernel(x, edge_index, edge_attr, op, W1, b1, W2, b2, Wp, bp, Wfc, bfc):
    src = edge_index[0]
    dst = edge_index[1]
    attr = edge_attr[:, 0]

    ew, deg_parts = _sc_deg(attr, dst)
    h1, hp1, dinv = _tc1(x, W1, deg_parts)
    agg1 = _sc_msg(hp1, src, dst, ew)
    h2, hp2 = _tc2(agg1, h1, dinv, W2, b1.reshape(1, _H))
    agg2 = _sc_msg(hp2, src, dst, ew)
    return _tc3(agg2, h2, dinv, op, Wp, bp.reshape(1, 128),
                Wfc, bfc.reshape(1, 1), b2.reshape(1, _H))


# speedup vs baseline: 17.3373x; 3.8502x over previous
import functools

import jax
import jax.numpy as jnp
from jax import lax
from jax.experimental import pallas as pl
from jax.experimental.pallas import tpu as pltpu
from jax.experimental.pallas import tpu_sc as plsc

_N = 10000
_E = 320000
_H = 64
_CHUNK = 128
_NCHUNKS = _E // _CHUNK
_GRP = 4
_NGRP = _NCHUNKS // _GRP
_NC = 2
_NS = 16
_NW = _NC * _NS
_BASE = _NCHUNKS // _NW
_REM = _NCHUNKS % _NW
_GBASE = _NGRP // _NW
_GREM = _NGRP % _NW
_RPS = _N // _NS


def _mesh():
    return plsc.VectorSubcoreMesh(core_axis_name="c", subcore_axis_name="s")


def _worker_id():
    return lax.axis_index("s") * _NC + lax.axis_index("c")


def _chunk_range(w):
    start = w * _BASE + jnp.minimum(w, _REM)
    count = _BASE + (w < _REM).astype(jnp.int32)
    return start, start + count


def _group_range(w):
    start = w * _GBASE + jnp.minimum(w, _GREM)
    count = _GBASE + (w < _GREM).astype(jnp.int32)
    return start, start + count


def _sc_deg_body(attr_hbm, dst_hbm, ew_hbm, deg_hbm, dst_v, attr_v, ew_v,
                 deg_local):
    w = _worker_id()

    @pl.loop(0, _N // 16)
    def _zero(i):
        deg_local[pl.ds(i * 16, 16)] = jnp.zeros((16,), jnp.float32)

    lo, hi = _chunk_range(w)

    @pl.loop(lo, hi)
    def _chunk(g):
        pltpu.sync_copy(dst_hbm.at[g], dst_v)
        pltpu.sync_copy(attr_hbm.at[g], attr_v)

        @pl.loop(0, _CHUNK // 16)
        def _grp(j):
            d16 = dst_v[pl.ds(j * 16, 16)]
            a16 = attr_v[pl.ds(j * 16, 16)]
            e16 = 1.0 / (a16 + 1.0)
            ew_v[pl.ds(j * 16, 16)] = e16
            plsc.addupdate_scatter(deg_local, [d16], e16)

        pltpu.sync_copy(ew_v, ew_hbm.at[g])

    pltpu.sync_copy(deg_local, deg_hbm.at[w])


def _sc_deg(attr2, dst2):
    kern = functools.partial(
        pl.kernel,
        compiler_params=pltpu.CompilerParams(needs_layout_passes=False, use_tc_tiling_on_sc=False),
        out_type=(
            jax.ShapeDtypeStruct((_NCHUNKS, _CHUNK), jnp.float32),
            jax.ShapeDtypeStruct((_NW, _N), jnp.float32),
        ),
        mesh=_mesh(),
        scratch_types=[
            pltpu.VMEM((_CHUNK,), jnp.int32),
            pltpu.VMEM((_CHUNK,), jnp.float32),
            pltpu.VMEM((_CHUNK,), jnp.float32),
            pltpu.VMEM((_N,), jnp.float32),
        ],
    )(_sc_deg_body)
    return kern(attr2, dst2)


def _sc_msg_body(hp_hbm, src_hbm, dst_hbm, ew_hbm, out_hbm, src_v, dst_v,
                 ew_v, rows_v, zbuf, acc_sh, sem):
    c = lax.axis_index("c")
    s = lax.axis_index("s")
    w = s * _NC + c

    @pl.loop(0, 125)
    def _zrow(i):
        for j in range(_H // 16):
            zbuf[i, pl.ds(j * 16, 16)] = jnp.zeros((16,), jnp.float32)

    for k in range(_RPS // 125):
        pltpu.sync_copy(zbuf, acc_sh.at[pl.ds(s * _RPS + k * 125, 125)])
    plsc.subcore_barrier()

    lo, hi = _group_range(w)

    @pl.loop(lo, hi)
    def _chunk(t):
        g4 = t * _GRP
        pltpu.sync_copy(src_hbm.at[pl.ds(g4, _GRP)], src_v)
        pltpu.sync_copy(dst_hbm.at[pl.ds(g4, _GRP)], dst_v)
        pltpu.sync_copy(ew_hbm.at[pl.ds(g4, _GRP)], ew_v)
        cps = [pltpu.async_copy(hp_hbm.at[src_v.at[j]], rows_v.at[j], sem)
               for j in range(_GRP)]
        for cp in cps:
            cp.wait()

        for j in range(_GRP):
            rj = rows_v.at[j]
            ej = ew_v.at[j]

            @pl.loop(0, _CHUNK, unroll=8)
            def _row(r):
                splat = plsc.load_gather(ej, [jnp.broadcast_to(r, (16,))])
                for k in range(_H // 16):
                    v = rj[r, pl.ds(k * 16, 16)]
                    rj[r, pl.ds(k * 16, 16)] = v * splat

        for j in range(_GRP):
            pltpu.sync_copy(rows_v.at[j], acc_sh.at[dst_v.at[j]], add=True)

    plsc.subcore_barrier()
    pltpu.sync_copy(acc_sh.at[pl.ds(s * _RPS, _RPS)],
                    out_hbm.at[c, pl.ds(s * _RPS, _RPS)])


def _sc_msg(hp, src2, dst2, ew2):
    kern = functools.partial(
        pl.kernel,
        compiler_params=pltpu.CompilerParams(needs_layout_passes=False, use_tc_tiling_on_sc=False),
        out_type=jax.ShapeDtypeStruct((_NC, _N, _H), jnp.float32),
        mesh=_mesh(),
        scratch_types=[
            pltpu.VMEM((_GRP, _CHUNK), jnp.int32),
            pltpu.VMEM((_GRP, _CHUNK), jnp.int32),
            pltpu.VMEM((_GRP, _CHUNK), jnp.float32),
            pltpu.VMEM((_GRP, _CHUNK, _H), jnp.float32),
            pltpu.VMEM((125, _H), jnp.float32),
            pltpu.VMEM_SHARED((_N, _H), jnp.float32),
            pltpu.SemaphoreType.DMA,
        ],
    )(_sc_msg_body)
    return kern(hp, src2, dst2, ew2)


def _tc1_body(x_ref, w1_ref, degp_ref, h1_ref, hp1_ref, dinv_ref):
    deg = jnp.sum(degp_ref[...], axis=0)[:, None] + 1.0
    dinv = jnp.where(deg > 0, lax.rsqrt(jnp.maximum(deg, 1e-12)), 0.0)
    h1 = jnp.dot(x_ref[...], w1_ref[...], preferred_element_type=jnp.float32)
    h1_ref[...] = h1
    hp1_ref[...] = h1 * dinv
    dinv_ref[...] = dinv


def _tc1(x, W1, deg_parts):
    return pl.pallas_call(
        _tc1_body,
        out_shape=(
            jax.ShapeDtypeStruct((_N, _H), jnp.float32),
            jax.ShapeDtypeStruct((_N, _H), jnp.float32),
            jax.ShapeDtypeStruct((_N, 1), jnp.float32),
        ),
    )(x, W1, deg_parts)


def _tc2_body(agg_ref, h1_ref, dinv_ref, w2_ref, b1_ref, h2_ref, hp2_ref):
    dinv = dinv_ref[...]
    a = agg_ref[...]
    z = dinv * (a[0] + a[1]) + (dinv * dinv) * h1_ref[...] + b1_ref[...]
    r = jnp.maximum(z, 0.0)
    h2 = jnp.dot(r, w2_ref[...], preferred_element_type=jnp.float32)
    h2_ref[...] = h2
    hp2_ref[...] = h2 * dinv


def _tc2(agg1, h1, dinv, W2, b1):
    return pl.pallas_call(
        _tc2_body,
        out_shape=(
            jax.ShapeDtypeStruct((_N, _H), jnp.float32),
            jax.ShapeDtypeStruct((_N, _H), jnp.float32),
        ),
    )(agg1, h1, dinv, W2, b1)


def _tc3_body(agg_ref, h2_ref, dinv_ref, op_ref, wp_ref, bp_ref, wfc_ref,
              bfc_ref, b2_ref, out_ref):
    dinv = dinv_ref[...]
    a = agg_ref[...]
    z = dinv * (a[0] + a[1]) + (dinv * dinv) * h2_ref[...] + b2_ref[...]
    r = jnp.maximum(z, 0.0)
    emb = jnp.dot(r, wp_ref[...], preferred_element_type=jnp.float32) \
        + bp_ref[...]
    wfc = wfc_ref[...]
    out = jnp.dot(emb, wfc[:128], preferred_element_type=jnp.float32) \
        + jnp.dot(op_ref[...], wfc[128:], preferred_element_type=jnp.float32) \
        + bfc_ref[...]
    out_ref[...] = out


def _tc3(agg2, h2, dinv, op, Wp, bp, Wfc, bfc, b2):
    return pl.pallas_call(
        _tc3_body,
        out_shape=jax.ShapeDtypeStruct((_N, 1), jnp.float32),
    )(agg2, h2, dinv, op, Wp, bp, Wfc, bfc, b2)


def kernel(x, edge_index, edge_attr, op, W1, b1, W2, b2, Wp, bp, Wfc, bfc):
    src = edge_index[0].reshape(_NCHUNKS, _CHUNK)
    dst = edge_index[1].reshape(_NCHUNKS, _CHUNK)
    attr = edge_attr[:, 0].reshape(_NCHUNKS, _CHUNK)

    ew, deg_parts = _sc_deg(attr, dst)
    h1, hp1, dinv = _tc1(x, W1, deg_parts)
    agg1 = _sc_msg(hp1, src, dst, ew)
    h2, hp2 = _tc2(agg1, h1, dinv, W2, b1.reshape(1, _H))
    agg2 = _sc_msg(hp2, src, dst, ew)
    return _tc3(agg2, h2, dinv, op, Wp, bp.reshape(1, 128),
                Wfc, bfc.reshape(1, 1), b2.reshape(1, _H))


# speedup vs baseline: 23.5152x; 1.3563x over previous
import functools

import jax
import jax.numpy as jnp
from jax import lax
from jax.experimental import pallas as pl
from jax.experimental.pallas import tpu as pltpu
from jax.experimental.pallas import tpu_sc as plsc

_N = 10000
_E = 320000
_H = 64
_CHUNK = 128
_NCHUNKS = _E // _CHUNK
_GRP = 4
_NGRP = _NCHUNKS // _GRP
_NC = 2
_NS = 16
_NW = _NC * _NS
_BASE = _NCHUNKS // _NW
_REM = _NCHUNKS % _NW
_GBASE = _NGRP // _NW
_GREM = _NGRP % _NW
_RPS = _N // _NS


def _mesh():
    return plsc.VectorSubcoreMesh(core_axis_name="c", subcore_axis_name="s")


def _worker_id():
    return lax.axis_index("s") * _NC + lax.axis_index("c")


def _chunk_range(w):
    start = w * _BASE + jnp.minimum(w, _REM)
    count = _BASE + (w < _REM).astype(jnp.int32)
    return start, start + count


def _group_range(w):
    start = w * _GBASE + jnp.minimum(w, _GREM)
    count = _GBASE + (w < _GREM).astype(jnp.int32)
    return start, start + count


def _sc_deg_body(attr_hbm, dst_hbm, ew_hbm, deg_hbm, dst_v, attr_v, ew_v,
                 deg_local):
    w = _worker_id()

    @pl.loop(0, _N // 16)
    def _zero(i):
        deg_local[pl.ds(i * 16, 16)] = jnp.zeros((16,), jnp.float32)

    lo, hi = _group_range(w)

    @pl.loop(lo, hi)
    def _chunk(t):
        g4 = t * _GRP
        pltpu.sync_copy(dst_hbm.at[pl.ds(g4, _GRP)], dst_v)
        pltpu.sync_copy(attr_hbm.at[pl.ds(g4, _GRP)], attr_v)

        for j in range(_GRP):

            @pl.loop(0, _CHUNK // 16)
            def _grp(i):
                d16 = dst_v[j, pl.ds(i * 16, 16)]
                a16 = attr_v[j, pl.ds(i * 16, 16)]
                e16 = 1.0 / (a16 + 1.0)
                ew_v[j, pl.ds(i * 16, 16)] = e16
                plsc.addupdate_scatter(deg_local, [d16], e16)

        pltpu.sync_copy(ew_v, ew_hbm.at[pl.ds(g4, _GRP)])

    pltpu.sync_copy(deg_local, deg_hbm.at[w])


def _sc_deg(attr2, dst2):
    kern = functools.partial(
        pl.kernel,
        compiler_params=pltpu.CompilerParams(needs_layout_passes=False, use_tc_tiling_on_sc=False),
        out_type=(
            jax.ShapeDtypeStruct((_NCHUNKS, _CHUNK), jnp.float32),
            jax.ShapeDtypeStruct((_NW, _N), jnp.float32),
        ),
        mesh=_mesh(),
        scratch_types=[
            pltpu.VMEM((_GRP, _CHUNK), jnp.int32),
            pltpu.VMEM((_GRP, _CHUNK), jnp.float32),
            pltpu.VMEM((_GRP, _CHUNK), jnp.float32),
            pltpu.VMEM((_N,), jnp.float32),
        ],
    )(_sc_deg_body)
    return kern(attr2, dst2)


def _sc_msg_body(hp_hbm, src_hbm, dst_hbm, ew_hbm, out_hbm, src_v, dst_v,
                 ew_v, rows_v, zbuf, acc_sh, gsem_a, gsem_b, ssem_a, ssem_b):
    c = lax.axis_index("c")
    s = lax.axis_index("s")
    w = s * _NC + c

    @pl.loop(0, 125)
    def _zrow(i):
        for j in range(_H // 16):
            zbuf[i, pl.ds(j * 16, 16)] = jnp.zeros((16,), jnp.float32)

    for k in range(_RPS // 125):
        pltpu.sync_copy(zbuf, acc_sh.at[pl.ds(s * _RPS + k * 125, 125)])
    plsc.subcore_barrier()

    lo, hi = _group_range(w)

    def load_edges(t, b):
        g4 = t * _GRP
        pltpu.sync_copy(src_hbm.at[pl.ds(g4, _GRP)], src_v.at[b])
        pltpu.sync_copy(dst_hbm.at[pl.ds(g4, _GRP)], dst_v.at[b])
        pltpu.sync_copy(ew_hbm.at[pl.ds(g4, _GRP)], ew_v.at[b])

    def issue_gather(b, sem):
        return [pltpu.async_copy(hp_hbm.at[src_v.at[b, j]],
                                 rows_v.at[b, j], sem)
                for j in range(_GRP)]

    def scale(b):
        for j in range(_GRP):
            rj = rows_v.at[b, j]
            ej = ew_v.at[b, j]

            @pl.loop(0, _CHUNK, unroll=8)
            def _row(r):
                splat = plsc.load_gather(ej, [jnp.broadcast_to(r, (16,))])
                for k in range(_H // 16):
                    v = rj[r, pl.ds(k * 16, 16)]
                    rj[r, pl.ds(k * 16, 16)] = v * splat

    def issue_scatter(b, sem):
        return [pltpu.async_copy(rows_v.at[b, j], acc_sh.at[dst_v.at[b, j]],
                                 sem, add=True)
                for j in range(_GRP)]

    @pl.loop(0, (hi - lo + 1) // 2)
    def _pair(i):
        t_a = lo + 2 * i
        t_b = t_a + 1
        b_valid = t_b < hi

        load_edges(t_a, 0)
        g_a = issue_gather(0, gsem_a)
        g_b = [pltpu.make_async_copy(hp_hbm.at[src_v.at[1, j]],
                                     rows_v.at[1, j], gsem_b)
               for j in range(_GRP)]

        @pl.when(b_valid)
        def _pre_b():
            load_edges(t_b, 1)
            for cp in g_b:
                cp.start()

        for cp in g_a:
            cp.wait()
        scale(0)
        s_a = issue_scatter(0, ssem_a)

        @pl.when(b_valid)
        def _run_b():
            for cp in g_b:
                cp.wait()
            scale(1)

        for cp in s_a:
            cp.wait()

        @pl.when(b_valid)
        def _post_b():
            s_b = issue_scatter(1, ssem_b)
            for cp in s_b:
                cp.wait()

    plsc.subcore_barrier()
    pltpu.sync_copy(acc_sh.at[pl.ds(s * _RPS, _RPS)],
                    out_hbm.at[c, pl.ds(s * _RPS, _RPS)])


def _sc_msg(hp, src2, dst2, ew2):
    kern = functools.partial(
        pl.kernel,
        compiler_params=pltpu.CompilerParams(needs_layout_passes=False, use_tc_tiling_on_sc=False),
        out_type=jax.ShapeDtypeStruct((_NC, _N, _H), jnp.float32),
        mesh=_mesh(),
        scratch_types=[
            pltpu.VMEM((2, _GRP, _CHUNK), jnp.int32),
            pltpu.VMEM((2, _GRP, _CHUNK), jnp.int32),
            pltpu.VMEM((2, _GRP, _CHUNK), jnp.float32),
            pltpu.VMEM((2, _GRP, _CHUNK, _H), jnp.float32),
            pltpu.VMEM((125, _H), jnp.float32),
            pltpu.VMEM_SHARED((_N, _H), jnp.float32),
            pltpu.SemaphoreType.DMA,
            pltpu.SemaphoreType.DMA,
            pltpu.SemaphoreType.DMA,
            pltpu.SemaphoreType.DMA,
        ],
    )(_sc_msg_body)
    return kern(hp, src2, dst2, ew2)


def _tc1_body(x_ref, w1_ref, degp_ref, h1_ref, hp1_ref, dinv_ref):
    deg = jnp.sum(degp_ref[...], axis=0)[:, None] + 1.0
    dinv = jnp.where(deg > 0, lax.rsqrt(jnp.maximum(deg, 1e-12)), 0.0)
    h1 = jnp.dot(x_ref[...], w1_ref[...], preferred_element_type=jnp.float32)
    h1_ref[...] = h1
    hp1_ref[...] = h1 * dinv
    dinv_ref[...] = dinv


def _tc1(x, W1, deg_parts):
    return pl.pallas_call(
        _tc1_body,
        out_shape=(
            jax.ShapeDtypeStruct((_N, _H), jnp.float32),
            jax.ShapeDtypeStruct((_N, _H), jnp.float32),
            jax.ShapeDtypeStruct((_N, 1), jnp.float32),
        ),
    )(x, W1, deg_parts)


def _tc2_body(agg_ref, h1_ref, dinv_ref, w2_ref, b1_ref, h2_ref, hp2_ref):
    dinv = dinv_ref[...]
    a = agg_ref[...]
    z = dinv * (a[0] + a[1]) + (dinv * dinv) * h1_ref[...] + b1_ref[...]
    r = jnp.maximum(z, 0.0)
    h2 = jnp.dot(r, w2_ref[...], preferred_element_type=jnp.float32)
    h2_ref[...] = h2
    hp2_ref[...] = h2 * dinv


def _tc2(agg1, h1, dinv, W2, b1):
    return pl.pallas_call(
        _tc2_body,
        out_shape=(
            jax.ShapeDtypeStruct((_N, _H), jnp.float32),
            jax.ShapeDtypeStruct((_N, _H), jnp.float32),
        ),
    )(agg1, h1, dinv, W2, b1)


def _tc3_body(agg_ref, h2_ref, dinv_ref, op_ref, wp_ref, bp_ref, wfc_ref,
              bfc_ref, b2_ref, out_ref):
    dinv = dinv_ref[...]
    a = agg_ref[...]
    z = dinv * (a[0] + a[1]) + (dinv * dinv) * h2_ref[...] + b2_ref[...]
    r = jnp.maximum(z, 0.0)
    emb = jnp.dot(r, wp_ref[...], preferred_element_type=jnp.float32) \
        + bp_ref[...]
    wfc = wfc_ref[...]
    out = jnp.dot(emb, wfc[:128], preferred_element_type=jnp.float32) \
        + jnp.dot(op_ref[...], wfc[128:], preferred_element_type=jnp.float32) \
        + bfc_ref[...]
    out_ref[...] = out


def _tc3(agg2, h2, dinv, op, Wp, bp, Wfc, bfc, b2):
    return pl.pallas_call(
        _tc3_body,
        out_shape=jax.ShapeDtypeStruct((_N, 1), jnp.float32),
    )(agg2, h2, dinv, op, Wp, bp, Wfc, bfc, b2)


def kernel(x, edge_index, edge_attr, op, W1, b1, W2, b2, Wp, bp, Wfc, bfc):
    src = edge_index[0].reshape(_NCHUNKS, _CHUNK)
    dst = edge_index[1].reshape(_NCHUNKS, _CHUNK)
    attr = edge_attr[:, 0].reshape(_NCHUNKS, _CHUNK)

    ew, deg_parts = _sc_deg(attr, dst)
    h1, hp1, dinv = _tc1(x, W1, deg_parts)
    agg1 = _sc_msg(hp1, src, dst, ew)
    h2, hp2 = _tc2(agg1, h1, dinv, W2, b1.reshape(1, _H))
    agg2 = _sc_msg(hp2, src, dst, ew)
    return _tc3(agg2, h2, dinv, op, Wp, bp.reshape(1, 128),
                Wfc, bfc.reshape(1, 1), b2.reshape(1, _H))


# speedup vs baseline: 25.7956x; 1.0970x over previous
import functools

import jax
import jax.numpy as jnp
from jax import lax
from jax.experimental import pallas as pl
from jax.experimental.pallas import tpu as pltpu
from jax.experimental.pallas import tpu_sc as plsc

_N = 10000
_E = 320000
_H = 64
_CHUNK = 128
_NCHUNKS = _E // _CHUNK
_GRP = 4
_NGRP = _NCHUNKS // _GRP
_MGRP = 2
_NMG = _NCHUNKS // _MGRP
_NSLOT = 4
_NC = 2
_NS = 16
_NW = _NC * _NS
_BASE = _NCHUNKS // _NW
_REM = _NCHUNKS % _NW
_GBASE = _NGRP // _NW
_GREM = _NGRP % _NW
_MBASE = _NMG // _NW
_MREM = _NMG % _NW
_RPS = _N // _NS


def _mesh():
    return plsc.VectorSubcoreMesh(core_axis_name="c", subcore_axis_name="s")


def _worker_id():
    return lax.axis_index("s") * _NC + lax.axis_index("c")


def _chunk_range(w):
    start = w * _BASE + jnp.minimum(w, _REM)
    count = _BASE + (w < _REM).astype(jnp.int32)
    return start, start + count


def _group_range(w):
    start = w * _GBASE + jnp.minimum(w, _GREM)
    count = _GBASE + (w < _GREM).astype(jnp.int32)
    return start, start + count


def _mgroup_range(w):
    start = w * _MBASE + jnp.minimum(w, _MREM)
    count = _MBASE + (w < _MREM).astype(jnp.int32)
    return start, start + count


def _sc_deg_body(attr_hbm, dst_hbm, ew_hbm, deg_hbm, dst_v, attr_v, ew_v,
                 deg_local):
    w = _worker_id()

    @pl.loop(0, _N // 16)
    def _zero(i):
        deg_local[pl.ds(i * 16, 16)] = jnp.zeros((16,), jnp.float32)

    lo, hi = _group_range(w)

    @pl.loop(lo, hi)
    def _chunk(t):
        g4 = t * _GRP
        pltpu.sync_copy(dst_hbm.at[pl.ds(g4, _GRP)], dst_v)
        pltpu.sync_copy(attr_hbm.at[pl.ds(g4, _GRP)], attr_v)

        for j in range(_GRP):

            @pl.loop(0, _CHUNK // 16)
            def _grp(i):
                d16 = dst_v[j, pl.ds(i * 16, 16)]
                a16 = attr_v[j, pl.ds(i * 16, 16)]
                e16 = 1.0 / (a16 + 1.0)
                ew_v[j, pl.ds(i * 16, 16)] = e16
                plsc.addupdate_scatter(deg_local, [d16], e16)

        pltpu.sync_copy(ew_v, ew_hbm.at[pl.ds(g4, _GRP)])

    pltpu.sync_copy(deg_local, deg_hbm.at[w])


def _sc_deg(attr2, dst2):
    kern = functools.partial(
        pl.kernel,
        compiler_params=pltpu.CompilerParams(needs_layout_passes=False, use_tc_tiling_on_sc=False),
        out_type=(
            jax.ShapeDtypeStruct((_NCHUNKS, _CHUNK), jnp.float32),
            jax.ShapeDtypeStruct((_NW, _N), jnp.float32),
        ),
        mesh=_mesh(),
        scratch_types=[
            pltpu.VMEM((_GRP, _CHUNK), jnp.int32),
            pltpu.VMEM((_GRP, _CHUNK), jnp.float32),
            pltpu.VMEM((_GRP, _CHUNK), jnp.float32),
            pltpu.VMEM((_N,), jnp.float32),
        ],
    )(_sc_deg_body)
    return kern(attr2, dst2)


def _sc_msg_body(hp_hbm, sd_hbm, ew_hbm, out_hbm, sd_v, ew_v, rows_v,
                 acc_sh, gsems, ssems):
    c = lax.axis_index("c")
    s = lax.axis_index("s")
    w = s * _NC + c

    @pl.loop(0, _CHUNK)
    def _zrow(i):
        for j in range(_H // 16):
            rows_v[0, 0, i, pl.ds(j * 16, 16)] = jnp.zeros((16,), jnp.float32)

    for k in range(_RPS // 125):
        pltpu.sync_copy(rows_v.at[0, 0, pl.ds(0, 125)],
                        acc_sh.at[pl.ds(s * _RPS + k * 125, 125)])
    plsc.subcore_barrier()

    lo, hi = _mgroup_range(w)

    def gather_cps(b):
        return [pltpu.make_async_copy(hp_hbm.at[sd_v.at[b, j, 0]],
                                      rows_v.at[b, j], gsems[b])
                for j in range(_MGRP)]

    def scatter_cps(b):
        return [pltpu.make_async_copy(rows_v.at[b, j],
                                      acc_sh.at[sd_v.at[b, j, 1]], ssems[b])
                for j in range(_MGRP)]

    def prefetch(t, b):
        g0 = t * _MGRP
        pltpu.sync_copy(sd_hbm.at[pl.ds(g0, _MGRP)], sd_v.at[b])
        pltpu.sync_copy(ew_hbm.at[pl.ds(g0, _MGRP)], ew_v.at[b])
        for cp in gather_cps(b):
            cp.start()

    def drain_scatter(b):
        for cp in scatter_cps(b):
            cp.wait()

    def scale(b):
        for j in range(_MGRP):
            rj = rows_v.at[b, j]
            ej = ew_v.at[b, j]

            @pl.loop(0, _CHUNK // 16)
            def _g16(g):
                ew16 = ej[pl.ds(g * 16, 16)]
                base = g * 16
                for r16 in range(16):
                    ridx = jnp.full((16,), r16, jnp.int32)
                    splat = ew16.at[ridx].get(mode="promise_in_bounds")
                    r = base + r16
                    for k in range(_H // 16):
                        v = rj[r, pl.ds(k * 16, 16)]
                        rj[r, pl.ds(k * 16, 16)] = v * splat

    def process(t, b):
        for cp in gather_cps(b):
            cp.wait()
        scale(b)
        for j in range(_MGRP):
            pltpu.async_copy(rows_v.at[b, j], acc_sh.at[sd_v.at[b, j, 1]],
                             ssems[b], add=True)

    for m in range(_NSLOT - 1):
        prefetch(lo + m, m)

    ntrip = (hi - lo + _NSLOT - 1) // _NSLOT

    @pl.loop(0, ntrip)
    def _trip(i):
        t0 = lo + _NSLOT * i
        for k in range(_NSLOT):
            tk = t0 + k
            pk = (k + _NSLOT - 1) % _NSLOT

            @pl.when(tk < hi)
            def _sub():
                process(tk, k)

                @pl.when(tk + _NSLOT - 1 < hi)
                def _pre():
                    if k == 0:
                        @pl.when(i > 0)
                        def _dr():
                            drain_scatter(pk)
                    else:
                        drain_scatter(pk)
                    prefetch(tk + _NSLOT - 1, pk)

    for b in range(_NSLOT):
        drain_scatter(b)

    plsc.subcore_barrier()
    pltpu.sync_copy(acc_sh.at[pl.ds(s * _RPS, _RPS)],
                    out_hbm.at[c, pl.ds(s * _RPS, _RPS)])


def _sc_msg(hp, sd2, ew2):
    kern = functools.partial(
        pl.kernel,
        compiler_params=pltpu.CompilerParams(needs_layout_passes=False, use_tc_tiling_on_sc=False),
        out_type=jax.ShapeDtypeStruct((_NC, _N, _H), jnp.float32),
        mesh=_mesh(),
        scratch_types=[
            pltpu.VMEM((_NSLOT, _MGRP, 2, _CHUNK), jnp.int32),
            pltpu.VMEM((_NSLOT, _MGRP, _CHUNK), jnp.float32),
            pltpu.VMEM((_NSLOT, _MGRP, _CHUNK, _H), jnp.float32),
            pltpu.VMEM_SHARED((_N, _H), jnp.float32),
            [pltpu.SemaphoreType.DMA] * _NSLOT,
            [pltpu.SemaphoreType.DMA] * _NSLOT,
        ],
    )(_sc_msg_body)
    return kern(hp, sd2, ew2)


def _tc1_body(x_ref, w1_ref, degp_ref, h1_ref, hp1_ref, dinv_ref):
    deg = jnp.sum(degp_ref[...], axis=0)[:, None] + 1.0
    dinv = jnp.where(deg > 0, lax.rsqrt(jnp.maximum(deg, 1e-12)), 0.0)
    h1 = jnp.dot(x_ref[...], w1_ref[...], preferred_element_type=jnp.float32)
    h1_ref[...] = h1
    hp1_ref[...] = h1 * dinv
    dinv_ref[...] = dinv


def _tc1(x, W1, deg_parts):
    return pl.pallas_call(
        _tc1_body,
        out_shape=(
            jax.ShapeDtypeStruct((_N, _H), jnp.float32),
            jax.ShapeDtypeStruct((_N, _H), jnp.float32),
            jax.ShapeDtypeStruct((_N, 1), jnp.float32),
        ),
    )(x, W1, deg_parts)


def _tc2_body(agg_ref, h1_ref, dinv_ref, w2_ref, b1_ref, h2_ref, hp2_ref):
    dinv = dinv_ref[...]
    a = agg_ref[...]
    z = dinv * (a[0] + a[1]) + (dinv * dinv) * h1_ref[...] + b1_ref[...]
    r = jnp.maximum(z, 0.0)
    h2 = jnp.dot(r, w2_ref[...], preferred_element_type=jnp.float32)
    h2_ref[...] = h2
    hp2_ref[...] = h2 * dinv


def _tc2(agg1, h1, dinv, W2, b1):
    return pl.pallas_call(
        _tc2_body,
        out_shape=(
            jax.ShapeDtypeStruct((_N, _H), jnp.float32),
            jax.ShapeDtypeStruct((_N, _H), jnp.float32),
        ),
    )(agg1, h1, dinv, W2, b1)


def _tc3_body(agg_ref, h2_ref, dinv_ref, op_ref, wp_ref, bp_ref, wfc_ref,
              bfc_ref, b2_ref, out_ref):
    dinv = dinv_ref[...]
    a = agg_ref[...]
    z = dinv * (a[0] + a[1]) + (dinv * dinv) * h2_ref[...] + b2_ref[...]
    r = jnp.maximum(z, 0.0)
    emb = jnp.dot(r, wp_ref[...], preferred_element_type=jnp.float32) \
        + bp_ref[...]
    wfc = wfc_ref[...]
    out = jnp.dot(emb, wfc[:128], preferred_element_type=jnp.float32) \
        + jnp.dot(op_ref[...], wfc[128:], preferred_element_type=jnp.float32) \
        + bfc_ref[...]
    out_ref[...] = out


def _tc3(agg2, h2, dinv, op, Wp, bp, Wfc, bfc, b2):
    return pl.pallas_call(
        _tc3_body,
        out_shape=jax.ShapeDtypeStruct((_N, 1), jnp.float32),
    )(agg2, h2, dinv, op, Wp, bp, Wfc, bfc, b2)


def kernel(x, edge_index, edge_attr, op, W1, b1, W2, b2, Wp, bp, Wfc, bfc):
    src = edge_index[0].reshape(_NCHUNKS, _CHUNK)
    dst = edge_index[1].reshape(_NCHUNKS, _CHUNK)
    attr = edge_attr[:, 0].reshape(_NCHUNKS, _CHUNK)
    sd = jnp.stack([src, dst], axis=1)

    ew, deg_parts = _sc_deg(attr, dst)
    h1, hp1, dinv = _tc1(x, W1, deg_parts)
    agg1 = _sc_msg(hp1, sd, ew)
    h2, hp2 = _tc2(agg1, h1, dinv, W2, b1.reshape(1, _H))
    agg2 = _sc_msg(hp2, sd, ew)
    return _tc3(agg2, h2, dinv, op, Wp, bp.reshape(1, 128),
                Wfc, bfc.reshape(1, 1), b2.reshape(1, _H))


# speedup vs baseline: 27.5168x; 1.0667x over previous
import functools

import jax
import jax.numpy as jnp
from jax import lax
from jax.experimental import pallas as pl
from jax.experimental.pallas import tpu as pltpu
from jax.experimental.pallas import tpu_sc as plsc

_N = 10000
_E = 320000
_H = 64
_CHUNK = 128
_NCHUNKS = _E // _CHUNK
_GRP = 4
_NGRP = _NCHUNKS // _GRP
_MGRP = 2
_NMG = _NCHUNKS // _MGRP
_NSLOT = 4
_NC = 2
_NS = 16
_NW = _NC * _NS
_BASE = _NCHUNKS // _NW
_REM = _NCHUNKS % _NW
_GBASE = _NGRP // _NW
_GREM = _NGRP % _NW
_MBASE = _NMG // _NW
_MREM = _NMG % _NW
_RPS = _N // _NS


def _mesh():
    return plsc.VectorSubcoreMesh(core_axis_name="c", subcore_axis_name="s")


def _worker_id():
    return lax.axis_index("s") * _NC + lax.axis_index("c")


def _chunk_range(w):
    start = w * _BASE + jnp.minimum(w, _REM)
    count = _BASE + (w < _REM).astype(jnp.int32)
    return start, start + count


def _group_range(w):
    start = w * _GBASE + jnp.minimum(w, _GREM)
    count = _GBASE + (w < _GREM).astype(jnp.int32)
    return start, start + count


def _mgroup_range(w):
    start = w * _MBASE + jnp.minimum(w, _MREM)
    count = _MBASE + (w < _MREM).astype(jnp.int32)
    return start, start + count


def _sc_deg_body(attr_hbm, dst_hbm, ew_hbm, deg_hbm, dst_v, attr_v, ew_v,
                 deg_local):
    w = _worker_id()

    @pl.loop(0, _N // 16)
    def _zero(i):
        deg_local[pl.ds(i * 16, 16)] = jnp.zeros((16,), jnp.float32)

    lo, hi = _group_range(w)

    @pl.loop(lo, hi)
    def _chunk(t):
        g4 = t * _GRP
        pltpu.sync_copy(dst_hbm.at[pl.ds(g4, _GRP)], dst_v)
        pltpu.sync_copy(attr_hbm.at[pl.ds(g4, _GRP)], attr_v)

        for j in range(_GRP):

            @pl.loop(0, _CHUNK // 16)
            def _grp(i):
                d16 = dst_v[j, pl.ds(i * 16, 16)]
                a16 = attr_v[j, pl.ds(i * 16, 16)]
                e16 = 1.0 / (a16 + 1.0)
                ew_v[j, pl.ds(i * 16, 16)] = e16
                plsc.addupdate_scatter(deg_local, [d16], e16)

        pltpu.sync_copy(ew_v, ew_hbm.at[pl.ds(g4, _GRP)])

    pltpu.sync_copy(deg_local, deg_hbm.at[w])


def _sc_deg(attr2, dst2):
    kern = functools.partial(
        pl.kernel,
        compiler_params=pltpu.CompilerParams(needs_layout_passes=False, use_tc_tiling_on_sc=False),
        out_type=(
            jax.ShapeDtypeStruct((_NCHUNKS, _CHUNK), jnp.float32),
            jax.ShapeDtypeStruct((_NW, _N), jnp.float32),
        ),
        mesh=_mesh(),
        scratch_types=[
            pltpu.VMEM((_GRP, _CHUNK), jnp.int32),
            pltpu.VMEM((_GRP, _CHUNK), jnp.float32),
            pltpu.VMEM((_GRP, _CHUNK), jnp.float32),
            pltpu.VMEM((_N,), jnp.float32),
        ],
    )(_sc_deg_body)
    return kern(attr2, dst2)


def _sc_msg_body(hp_hbm, pk_hbm, out_hbm, sd_v, rows_v, acc_sh, gsems,
                 ssems):
    c = lax.axis_index("c")
    s = lax.axis_index("s")
    w = s * _NC + c

    @pl.loop(0, _CHUNK)
    def _zrow(i):
        for j in range(_H // 16):
            rows_v[0, 0, i, pl.ds(j * 16, 16)] = jnp.zeros((16,), jnp.float32)

    for k in range(_RPS // 125):
        pltpu.sync_copy(rows_v.at[0, 0, pl.ds(0, 125)],
                        acc_sh.at[pl.ds(s * _RPS + k * 125, 125)])
    plsc.subcore_barrier()

    lo, hi = _mgroup_range(w)

    def gather_cps(b):
        return [pltpu.make_async_copy(hp_hbm.at[sd_v.at[b, j]],
                                      rows_v.at[b, j], gsems[b])
                for j in range(_MGRP)]

    def scatter_cps(b):
        return [pltpu.make_async_copy(rows_v.at[b, j],
                                      acc_sh.at[sd_v.at[b, _MGRP + j]],
                                      ssems[b])
                for j in range(_MGRP)]

    def prefetch(t, b):
        pltpu.sync_copy(pk_hbm.at[t], sd_v.at[b])
        for cp in gather_cps(b):
            cp.start()

    def drain_scatter(b):
        for cp in scatter_cps(b):
            cp.wait()

    def scale(b):
        for j in range(_MGRP):
            rj = rows_v.at[b, j]
            ej = sd_v.at[b, 2 * _MGRP + j]

            @pl.loop(0, _CHUNK // 16)
            def _g16(g):
                ew16 = plsc.bitcast(ej[pl.ds(g * 16, 16)], jnp.float32)
                base = g * 16
                for r16 in range(16):
                    ridx = jnp.full((16,), r16, jnp.int32)
                    splat = ew16.at[ridx].get(mode="promise_in_bounds")
                    r = base + r16
                    for k in range(_H // 16):
                        v = rj[r, pl.ds(k * 16, 16)]
                        rj[r, pl.ds(k * 16, 16)] = v * splat

    def process(t, b):
        for cp in gather_cps(b):
            cp.wait()
        scale(b)
        for j in range(_MGRP):
            pltpu.async_copy(rows_v.at[b, j],
                             acc_sh.at[sd_v.at[b, _MGRP + j]],
                             ssems[b], add=True)

    for m in range(_NSLOT - 1):
        prefetch(lo + m, m)

    ntrip = (hi - lo + _NSLOT - 1) // _NSLOT

    @pl.loop(0, ntrip)
    def _trip(i):
        t0 = lo + _NSLOT * i
        for k in range(_NSLOT):
            tk = t0 + k
            pk = (k + _NSLOT - 1) % _NSLOT

            @pl.when(tk < hi)
            def _sub():
                process(tk, k)

                @pl.when(tk + _NSLOT - 1 < hi)
                def _pre():
                    if k == 0:
                        @pl.when(i > 0)
                        def _dr():
                            drain_scatter(pk)
                    else:
                        drain_scatter(pk)
                    prefetch(tk + _NSLOT - 1, pk)

    for b in range(_NSLOT):
        drain_scatter(b)

    plsc.subcore_barrier()
    pltpu.sync_copy(acc_sh.at[pl.ds(s * _RPS, _RPS)],
                    out_hbm.at[c, pl.ds(s * _RPS, _RPS)])


def _sc_msg(hp, pk):
    kern = functools.partial(
        pl.kernel,
        compiler_params=pltpu.CompilerParams(needs_layout_passes=False, use_tc_tiling_on_sc=False),
        out_type=jax.ShapeDtypeStruct((_NC, _N, _H), jnp.float32),
        mesh=_mesh(),
        scratch_types=[
            pltpu.VMEM((_NSLOT, 3 * _MGRP, _CHUNK), jnp.int32),
            pltpu.VMEM((_NSLOT, _MGRP, _CHUNK, _H), jnp.float32),
            pltpu.VMEM_SHARED((_N, _H), jnp.float32),
            [pltpu.SemaphoreType.DMA] * _NSLOT,
            [pltpu.SemaphoreType.DMA] * _NSLOT,
        ],
    )(_sc_msg_body)
    return kern(hp, pk)


def _tc1_body(x_ref, w1_ref, degp_ref, h1_ref, hp1_ref, dinv_ref):
    deg = jnp.sum(degp_ref[...], axis=0)[:, None] + 1.0
    dinv = jnp.where(deg > 0, lax.rsqrt(jnp.maximum(deg, 1e-12)), 0.0)
    h1 = jnp.dot(x_ref[...], w1_ref[...], preferred_element_type=jnp.float32)
    h1_ref[...] = h1
    hp1_ref[...] = h1 * dinv
    dinv_ref[...] = dinv


def _tc1(x, W1, deg_parts):
    return pl.pallas_call(
        _tc1_body,
        out_shape=(
            jax.ShapeDtypeStruct((_N, _H), jnp.float32),
            jax.ShapeDtypeStruct((_N, _H), jnp.float32),
            jax.ShapeDtypeStruct((_N, 1), jnp.float32),
        ),
    )(x, W1, deg_parts)


def _tc2_body(agg_ref, h1_ref, dinv_ref, w2_ref, b1_ref, h2_ref, hp2_ref):
    dinv = dinv_ref[...]
    a = agg_ref[...]
    z = dinv * (a[0] + a[1]) + (dinv * dinv) * h1_ref[...] + b1_ref[...]
    r = jnp.maximum(z, 0.0)
    h2 = jnp.dot(r, w2_ref[...], preferred_element_type=jnp.float32)
    h2_ref[...] = h2
    hp2_ref[...] = h2 * dinv


def _tc2(agg1, h1, dinv, W2, b1):
    return pl.pallas_call(
        _tc2_body,
        out_shape=(
            jax.ShapeDtypeStruct((_N, _H), jnp.float32),
            jax.ShapeDtypeStruct((_N, _H), jnp.float32),
        ),
    )(agg1, h1, dinv, W2, b1)


def _tc3_body(agg_ref, h2_ref, dinv_ref, op_ref, wp_ref, bp_ref, wfc_ref,
              bfc_ref, b2_ref, out_ref):
    dinv = dinv_ref[...]
    a = agg_ref[...]
    z = dinv * (a[0] + a[1]) + (dinv * dinv) * h2_ref[...] + b2_ref[...]
    r = jnp.maximum(z, 0.0)
    emb = jnp.dot(r, wp_ref[...], preferred_element_type=jnp.float32) \
        + bp_ref[...]
    wfc = wfc_ref[...]
    out = jnp.dot(emb, wfc[:128], preferred_element_type=jnp.float32) \
        + jnp.dot(op_ref[...], wfc[128:], preferred_element_type=jnp.float32) \
        + bfc_ref[...]
    out_ref[...] = out


def _tc3(agg2, h2, dinv, op, Wp, bp, Wfc, bfc, b2):
    return pl.pallas_call(
        _tc3_body,
        out_shape=jax.ShapeDtypeStruct((_N, 1), jnp.float32),
    )(agg2, h2, dinv, op, Wp, bp, Wfc, bfc, b2)


def kernel(x, edge_index, edge_attr, op, W1, b1, W2, b2, Wp, bp, Wfc, bfc):
    src = edge_index[0].reshape(_NMG, _MGRP, _CHUNK)
    dst = edge_index[1].reshape(_NMG, _MGRP, _CHUNK)
    attr = edge_attr[:, 0].reshape(_NCHUNKS, _CHUNK)

    ew, deg_parts = _sc_deg(attr, dst.reshape(_NCHUNKS, _CHUNK))
    ewi = jax.lax.bitcast_convert_type(ew.reshape(_NMG, _MGRP, _CHUNK),
                                       jnp.int32)
    pk = jnp.concatenate([src, dst, ewi], axis=1)

    h1, hp1, dinv = _tc1(x, W1, deg_parts)
    agg1 = _sc_msg(hp1, pk)
    h2, hp2 = _tc2(agg1, h1, dinv, W2, b1.reshape(1, _H))
    agg2 = _sc_msg(hp2, pk)
    return _tc3(agg2, h2, dinv, op, Wp, bp.reshape(1, 128),
                Wfc, bfc.reshape(1, 1), b2.reshape(1, _H))


# speedup vs baseline: 27.5658x; 1.0018x over previous
import functools

import jax
import jax.numpy as jnp
from jax import lax
from jax.experimental import pallas as pl
from jax.experimental.pallas import tpu as pltpu
from jax.experimental.pallas import tpu_sc as plsc

_N = 10000
_E = 320000
_H = 64
_CHUNK = 128
_NCHUNKS = _E // _CHUNK
_GRP = 4
_NGRP = _NCHUNKS // _GRP
_MGRP = 2
_NMG = _NCHUNKS // _MGRP
_NSLOT = 4
_NC = 2
_NS = 16
_NW = _NC * _NS
_BASE = _NCHUNKS // _NW
_REM = _NCHUNKS % _NW
_GBASE = _NGRP // _NW
_GREM = _NGRP % _NW
_MBASE = _NMG // _NW
_MREM = _NMG % _NW
_RPS = _N // _NS


def _mesh():
    return plsc.VectorSubcoreMesh(core_axis_name="c", subcore_axis_name="s")


def _worker_id():
    return lax.axis_index("s") * _NC + lax.axis_index("c")


def _chunk_range(w):
    start = w * _BASE + jnp.minimum(w, _REM)
    count = _BASE + (w < _REM).astype(jnp.int32)
    return start, start + count


def _group_range(w):
    start = w * _GBASE + jnp.minimum(w, _GREM)
    count = _GBASE + (w < _GREM).astype(jnp.int32)
    return start, start + count


def _mgroup_range(w):
    start = w * _MBASE + jnp.minimum(w, _MREM)
    count = _MBASE + (w < _MREM).astype(jnp.int32)
    return start, start + count


def _sc_deg_body(attr_hbm, dst_hbm, ew_hbm, deg_hbm, dst_v, attr_v, ew_v,
                 deg_local):
    w = _worker_id()

    @pl.loop(0, _N // 16)
    def _zero(i):
        deg_local[pl.ds(i * 16, 16)] = jnp.zeros((16,), jnp.float32)

    lo, hi = _group_range(w)

    @pl.loop(lo, hi)
    def _chunk(t):
        g4 = t * _GRP
        pltpu.sync_copy(dst_hbm.at[pl.ds(g4, _GRP)], dst_v)
        pltpu.sync_copy(attr_hbm.at[pl.ds(g4, _GRP)], attr_v)

        for j in range(_GRP):

            @pl.loop(0, _CHUNK // 16)
            def _grp(i):
                d16 = dst_v[j, pl.ds(i * 16, 16)]
                a16 = attr_v[j, pl.ds(i * 16, 16)]
                e16 = 1.0 / (a16 + 1.0)
                ew_v[j, pl.ds(i * 16, 16)] = e16
                plsc.addupdate_scatter(deg_local, [d16], e16)

        pltpu.sync_copy(ew_v, ew_hbm.at[pl.ds(g4, _GRP)])

    pltpu.sync_copy(deg_local, deg_hbm.at[w])


def _sc_deg(attr2, dst2):
    kern = functools.partial(
        pl.kernel,
        compiler_params=pltpu.CompilerParams(needs_layout_passes=False, use_tc_tiling_on_sc=False),
        out_type=(
            jax.ShapeDtypeStruct((_NCHUNKS, _CHUNK), jnp.float32),
            jax.ShapeDtypeStruct((_NW, _N), jnp.float32),
        ),
        mesh=_mesh(),
        scratch_types=[
            pltpu.VMEM((_GRP, _CHUNK), jnp.int32),
            pltpu.VMEM((_GRP, _CHUNK), jnp.float32),
            pltpu.VMEM((_GRP, _CHUNK), jnp.float32),
            pltpu.VMEM((_N,), jnp.float32),
        ],
    )(_sc_deg_body)
    return kern(attr2, dst2)


def _sc_msg_body(hp_hbm, pk_hbm, out_hbm, sd_v, rows_v, acc_sh, gsems,
                 ssems):
    c = lax.axis_index("c")
    s = lax.axis_index("s")
    w = s * _NC + c

    @pl.loop(0, _CHUNK)
    def _zrow(i):
        for j in range(_H // 16):
            rows_v[0, 0, i, pl.ds(j * 16, 16)] = jnp.zeros((16,), jnp.float32)

    for k in range(_RPS // 125):
        pltpu.sync_copy(rows_v.at[0, 0, pl.ds(0, 125)],
                        acc_sh.at[pl.ds(s * _RPS + k * 125, 125)])
    plsc.subcore_barrier()

    lo, hi = _mgroup_range(w)

    def gather_cps(b):
        return [pltpu.make_async_copy(hp_hbm.at[sd_v.at[b, j]],
                                      rows_v.at[b, j], gsems[b])
                for j in range(_MGRP)]

    def scatter_cps(b):
        return [pltpu.make_async_copy(rows_v.at[b, j],
                                      acc_sh.at[sd_v.at[b, _MGRP + j]],
                                      ssems[b])
                for j in range(_MGRP)]

    def prefetch(t, b):
        pltpu.sync_copy(pk_hbm.at[t], sd_v.at[b])
        for cp in gather_cps(b):
            cp.start()

    def drain_scatter(b):
        for cp in scatter_cps(b):
            cp.wait()

    def scale(b):
        for j in range(_MGRP):
            rj = rows_v.at[b, j]
            ej = sd_v.at[b, 2 * _MGRP + j]

            @pl.loop(0, _CHUNK // 16)
            def _g16(g):
                ew16 = plsc.bitcast(ej[pl.ds(g * 16, 16)], jnp.float32)
                base = g * 16
                for r16 in range(16):
                    ridx = jnp.full((16,), r16, jnp.int32)
                    splat = ew16.at[ridx].get(mode="promise_in_bounds")
                    r = base + r16
                    for k in range(_H // 16):
                        v = rj[r, pl.ds(k * 16, 16)]
                        rj[r, pl.ds(k * 16, 16)] = v * splat

    def process(t, b):
        for cp in gather_cps(b):
            cp.wait()
        scale(b)
        for j in range(_MGRP):
            pltpu.async_copy(rows_v.at[b, j],
                             acc_sh.at[sd_v.at[b, _MGRP + j]],
                             ssems[b], add=True)

    for m in range(_NSLOT - 1):
        prefetch(lo + m, m)

    ntrip = (hi - lo + _NSLOT - 1) // _NSLOT

    @pl.loop(0, ntrip)
    def _trip(i):
        t0 = lo + _NSLOT * i
        for k in range(_NSLOT):
            tk = t0 + k
            pk = (k + _NSLOT - 1) % _NSLOT

            @pl.when(tk < hi)
            def _sub():
                process(tk, k)

                @pl.when(tk + _NSLOT - 1 < hi)
                def _pre():
                    if k == 0:
                        @pl.when(i > 0)
                        def _dr():
                            drain_scatter(pk)
                    else:
                        drain_scatter(pk)
                    prefetch(tk + _NSLOT - 1, pk)

    for b in range(_NSLOT):
        drain_scatter(b)

    plsc.subcore_barrier()
    pltpu.sync_copy(acc_sh.at[pl.ds(s * _RPS, _RPS)],
                    out_hbm.at[c, pl.ds(s * _RPS, _RPS)])


def _sc_msg(hp, pk):
    kern = functools.partial(
        pl.kernel,
        compiler_params=pltpu.CompilerParams(needs_layout_passes=False, use_tc_tiling_on_sc=False),
        out_type=jax.ShapeDtypeStruct((_NC, _N, _H), jnp.float32),
        mesh=_mesh(),
        scratch_types=[
            pltpu.VMEM((_NSLOT, 3 * _MGRP, _CHUNK), jnp.int32),
            pltpu.VMEM((_NSLOT, _MGRP, _CHUNK, _H), jnp.float32),
            pltpu.VMEM_SHARED((_N, _H), jnp.float32),
            [pltpu.SemaphoreType.DMA] * _NSLOT,
            [pltpu.SemaphoreType.DMA] * _NSLOT,
        ],
    )(_sc_msg_body)
    return kern(hp, pk)


def _tca_body(x_ref, w1_ref, h1_ref):
    h1_ref[...] = jnp.dot(x_ref[...], w1_ref[...],
                          preferred_element_type=jnp.float32)


def _tca(x, W1):
    return pl.pallas_call(
        _tca_body,
        out_shape=jax.ShapeDtypeStruct((_N, _H), jnp.float32),
    )(x, W1)


def _tcb_body(h1_ref, degp_ref, hp1_ref, dinv_ref):
    deg = jnp.sum(degp_ref[...], axis=0)[:, None] + 1.0
    dinv = jnp.where(deg > 0, lax.rsqrt(jnp.maximum(deg, 1e-12)), 0.0)
    hp1_ref[...] = h1_ref[...] * dinv
    dinv_ref[...] = dinv


def _tcb(h1, deg_parts):
    return pl.pallas_call(
        _tcb_body,
        out_shape=(
            jax.ShapeDtypeStruct((_N, _H), jnp.float32),
            jax.ShapeDtypeStruct((_N, 1), jnp.float32),
        ),
    )(h1, deg_parts)


def _tc2_body(agg_ref, h1_ref, dinv_ref, w2_ref, b1_ref, h2_ref, hp2_ref):
    dinv = dinv_ref[...]
    a = agg_ref[...]
    z = dinv * (a[0] + a[1]) + (dinv * dinv) * h1_ref[...] + b1_ref[...]
    r = jnp.maximum(z, 0.0)
    h2 = jnp.dot(r, w2_ref[...], preferred_element_type=jnp.float32)
    h2_ref[...] = h2
    hp2_ref[...] = h2 * dinv


def _tc2(agg1, h1, dinv, W2, b1):
    return pl.pallas_call(
        _tc2_body,
        out_shape=(
            jax.ShapeDtypeStruct((_N, _H), jnp.float32),
            jax.ShapeDtypeStruct((_N, _H), jnp.float32),
        ),
    )(agg1, h1, dinv, W2, b1)


def _tc3_body(agg_ref, h2_ref, dinv_ref, op_ref, wp_ref, bp_ref, wfc_ref,
              bfc_ref, b2_ref, out_ref):
    dinv = dinv_ref[...]
    a = agg_ref[...]
    z = dinv * (a[0] + a[1]) + (dinv * dinv) * h2_ref[...] + b2_ref[...]
    r = jnp.maximum(z, 0.0)
    emb = jnp.dot(r, wp_ref[...], preferred_element_type=jnp.float32) \
        + bp_ref[...]
    wfc = wfc_ref[...]
    out = jnp.dot(emb, wfc[:128], preferred_element_type=jnp.float32) \
        + jnp.dot(op_ref[...], wfc[128:], preferred_element_type=jnp.float32) \
        + bfc_ref[...]
    out_ref[...] = out


def _tc3(agg2, h2, dinv, op, Wp, bp, Wfc, bfc, b2):
    return pl.pallas_call(
        _tc3_body,
        out_shape=jax.ShapeDtypeStruct((_N, 1), jnp.float32),
    )(agg2, h2, dinv, op, Wp, bp, Wfc, bfc, b2)


def kernel(x, edge_index, edge_attr, op, W1, b1, W2, b2, Wp, bp, Wfc, bfc):
    src = edge_index[0].reshape(_NMG, _MGRP, _CHUNK)
    dst = edge_index[1].reshape(_NMG, _MGRP, _CHUNK)
    attr = edge_attr[:, 0].reshape(_NCHUNKS, _CHUNK)

    ew, deg_parts = _sc_deg(attr, dst.reshape(_NCHUNKS, _CHUNK))
    ewi = jax.lax.bitcast_convert_type(ew.reshape(_NMG, _MGRP, _CHUNK),
                                       jnp.int32)
    pk = jnp.concatenate([src, dst, ewi], axis=1)

    h1 = _tca(x, W1)
    hp1, dinv = _tcb(h1, deg_parts)
    agg1 = _sc_msg(hp1, pk)
    h2, hp2 = _tc2(agg1, h1, dinv, W2, b1.reshape(1, _H))
    agg2 = _sc_msg(hp2, pk)
    return _tc3(agg2, h2, dinv, op, Wp, bp.reshape(1, 128),
                Wfc, bfc.reshape(1, 1), b2.reshape(1, _H))


# speedup vs baseline: 27.9753x; 1.0149x over previous
import functools

import jax
import jax.numpy as jnp
from jax import lax
from jax.experimental import pallas as pl
from jax.experimental.pallas import tpu as pltpu
from jax.experimental.pallas import tpu_sc as plsc

_N = 10000
_E = 320000
_H = 64
_CHUNK = 128
_NCHUNKS = _E // _CHUNK
_GRP = 4
_NGRP = _NCHUNKS // _GRP
_MGRP = 2
_NMG = _NCHUNKS // _MGRP
_NSLOT = 4
_NC = 2
_NS = 16
_NW = _NC * _NS
_BASE = _NCHUNKS // _NW
_REM = _NCHUNKS % _NW
_GBASE = _NGRP // _NW
_GREM = _NGRP % _NW
_MBASE = _NMG // _NW
_MREM = _NMG % _NW
_RPS = _N // _NS


def _mesh():
    return plsc.VectorSubcoreMesh(core_axis_name="c", subcore_axis_name="s")


def _worker_id():
    return lax.axis_index("s") * _NC + lax.axis_index("c")


def _chunk_range(w):
    start = w * _BASE + jnp.minimum(w, _REM)
    count = _BASE + (w < _REM).astype(jnp.int32)
    return start, start + count


def _group_range(w):
    start = w * _GBASE + jnp.minimum(w, _GREM)
    count = _GBASE + (w < _GREM).astype(jnp.int32)
    return start, start + count


def _mgroup_range(w):
    start = w * _MBASE + jnp.minimum(w, _MREM)
    count = _MBASE + (w < _MREM).astype(jnp.int32)
    return start, start + count


def _sc_deg_body(attr_hbm, dst_hbm, ew_hbm, deg_hbm, dst_v, attr_v, ew_v,
                 deg_local):
    w = _worker_id()

    @pl.loop(0, _N // 16)
    def _zero(i):
        deg_local[pl.ds(i * 16, 16)] = jnp.zeros((16,), jnp.float32)

    lo, hi = _group_range(w)

    @pl.loop(lo, hi)
    def _chunk(t):
        g4 = t * _GRP
        pltpu.sync_copy(dst_hbm.at[pl.ds(g4, _GRP)], dst_v)
        pltpu.sync_copy(attr_hbm.at[pl.ds(g4, _GRP)], attr_v)

        for j in range(_GRP):

            @pl.loop(0, _CHUNK // 16)
            def _grp(i):
                d16 = dst_v[j, pl.ds(i * 16, 16)]
                a16 = attr_v[j, pl.ds(i * 16, 16)]
                e16 = 1.0 / (a16 + 1.0)
                ew_v[j, pl.ds(i * 16, 16)] = e16
                plsc.addupdate_scatter(deg_local, [d16], e16)

        pltpu.sync_copy(ew_v, ew_hbm.at[pl.ds(g4, _GRP)])

    pltpu.sync_copy(deg_local, deg_hbm.at[w])


def _sc_deg(attr2, dst2):
    kern = functools.partial(
        pl.kernel,
        compiler_params=pltpu.CompilerParams(needs_layout_passes=False, use_tc_tiling_on_sc=False),
        out_type=(
            jax.ShapeDtypeStruct((_NCHUNKS, _CHUNK), jnp.float32),
            jax.ShapeDtypeStruct((_NW, _N), jnp.float32),
        ),
        mesh=_mesh(),
        scratch_types=[
            pltpu.VMEM((_GRP, _CHUNK), jnp.int32),
            pltpu.VMEM((_GRP, _CHUNK), jnp.float32),
            pltpu.VMEM((_GRP, _CHUNK), jnp.float32),
            pltpu.VMEM((_N,), jnp.float32),
        ],
    )(_sc_deg_body)
    return kern(attr2, dst2)


def _sc_msg_body(hp_hbm, pk_hbm, out_hbm, sd_v, rows_v, acc_sh, gsems,
                 ssems, esems):
    c = lax.axis_index("c")
    s = lax.axis_index("s")
    w = s * _NC + c

    @pl.loop(0, _CHUNK)
    def _zrow(i):
        for j in range(_H // 16):
            rows_v[0, 0, i, pl.ds(j * 16, 16)] = jnp.zeros((16,), jnp.float32)

    for k in range(_RPS // 125):
        pltpu.sync_copy(rows_v.at[0, 0, pl.ds(0, 125)],
                        acc_sh.at[pl.ds(s * _RPS + k * 125, 125)])
    plsc.subcore_barrier()

    lo, hi = _mgroup_range(w)

    def gather_cps(b):
        return [pltpu.make_async_copy(hp_hbm.at[sd_v.at[b, j]],
                                      rows_v.at[b, j], gsems[b])
                for j in range(_MGRP)]

    def scatter_cps(b):
        return [pltpu.make_async_copy(rows_v.at[b, j],
                                      acc_sh.at[sd_v.at[b, _MGRP + j]],
                                      ssems[b])
                for j in range(_MGRP)]

    def start_edges(t, b):
        pltpu.make_async_copy(pk_hbm.at[t], sd_v.at[b], esems[b]).start()

    def start_gather(b):
        pltpu.make_async_copy(pk_hbm.at[0], sd_v.at[b], esems[b]).wait()
        for cp in gather_cps(b):
            cp.start()

    def drain_scatter(b):
        for cp in scatter_cps(b):
            cp.wait()

    def scale(b):
        for j in range(_MGRP):
            rj = rows_v.at[b, j]
            ej = sd_v.at[b, 2 * _MGRP + j]

            @pl.loop(0, _CHUNK // 16)
            def _g16(g):
                ew16 = plsc.bitcast(ej[pl.ds(g * 16, 16)], jnp.float32)
                base = g * 16
                for r16 in range(16):
                    ridx = jnp.full((16,), r16, jnp.int32)
                    splat = ew16.at[ridx].get(mode="promise_in_bounds")
                    r = base + r16
                    for k in range(_H // 16):
                        v = rj[r, pl.ds(k * 16, 16)]
                        rj[r, pl.ds(k * 16, 16)] = v * splat

    def process(t, b):
        for cp in gather_cps(b):
            cp.wait()
        scale(b)
        for j in range(_MGRP):
            pltpu.async_copy(rows_v.at[b, j],
                             acc_sh.at[sd_v.at[b, _MGRP + j]],
                             ssems[b], add=True)

    for m in range(_NSLOT - 1):
        start_edges(lo + m, m)
    for m in range(_NSLOT - 2):
        start_gather(m)

    ntrip = (hi - lo + _NSLOT - 1) // _NSLOT

    @pl.loop(0, ntrip)
    def _trip(i):
        t0 = lo + _NSLOT * i
        for k in range(_NSLOT):
            tk = t0 + k
            pg = (k + _NSLOT - 2) % _NSLOT
            pe = (k + _NSLOT - 1) % _NSLOT

            @pl.when(tk < hi)
            def _sub():
                process(tk, k)

                @pl.when(tk + _NSLOT - 2 < hi)
                def _preg():
                    start_gather(pg)

                @pl.when(tk + _NSLOT - 1 < hi)
                def _pree():
                    if k == 0:
                        @pl.when(i > 0)
                        def _dr():
                            drain_scatter(pe)
                    else:
                        drain_scatter(pe)
                    start_edges(tk + _NSLOT - 1, pe)

    for b in range(_NSLOT):
        drain_scatter(b)

    plsc.subcore_barrier()
    pltpu.sync_copy(acc_sh.at[pl.ds(s * _RPS, _RPS)],
                    out_hbm.at[c, pl.ds(s * _RPS, _RPS)])


def _sc_msg(hp, pk):
    kern = functools.partial(
        pl.kernel,
        compiler_params=pltpu.CompilerParams(needs_layout_passes=False, use_tc_tiling_on_sc=False),
        out_type=jax.ShapeDtypeStruct((_NC, _N, _H), jnp.float32),
        mesh=_mesh(),
        scratch_types=[
            pltpu.VMEM((_NSLOT, 3 * _MGRP, _CHUNK), jnp.int32),
            pltpu.VMEM((_NSLOT, _MGRP, _CHUNK, _H), jnp.float32),
            pltpu.VMEM_SHARED((_N, _H), jnp.float32),
            [pltpu.SemaphoreType.DMA] * _NSLOT,
            [pltpu.SemaphoreType.DMA] * _NSLOT,
            [pltpu.SemaphoreType.DMA] * _NSLOT,
        ],
    )(_sc_msg_body)
    return kern(hp, pk)


def _tca_body(x_ref, w1_ref, h1_ref):
    h1_ref[...] = jnp.dot(x_ref[...], w1_ref[...],
                          preferred_element_type=jnp.float32)


def _tca(x, W1):
    return pl.pallas_call(
        _tca_body,
        out_shape=jax.ShapeDtypeStruct((_N, _H), jnp.float32),
    )(x, W1)


def _tcb_body(h1_ref, degp_ref, hp1_ref, dinv_ref):
    deg = jnp.sum(degp_ref[...], axis=0)[:, None] + 1.0
    dinv = jnp.where(deg > 0, lax.rsqrt(jnp.maximum(deg, 1e-12)), 0.0)
    hp1_ref[...] = h1_ref[...] * dinv
    dinv_ref[...] = dinv


def _tcb(h1, deg_parts):
    return pl.pallas_call(
        _tcb_body,
        out_shape=(
            jax.ShapeDtypeStruct((_N, _H), jnp.float32),
            jax.ShapeDtypeStruct((_N, 1), jnp.float32),
        ),
    )(h1, deg_parts)


def _tc2_body(agg_ref, h1_ref, dinv_ref, w2_ref, b1_ref, h2_ref, hp2_ref):
    dinv = dinv_ref[...]
    a = agg_ref[...]
    z = dinv * (a[0] + a[1]) + (dinv * dinv) * h1_ref[...] + b1_ref[...]
    r = jnp.maximum(z, 0.0)
    h2 = jnp.dot(r, w2_ref[...], preferred_element_type=jnp.float32)
    h2_ref[...] = h2
    hp2_ref[...] = h2 * dinv


def _tc2(agg1, h1, dinv, W2, b1):
    return pl.pallas_call(
        _tc2_body,
        out_shape=(
            jax.ShapeDtypeStruct((_N, _H), jnp.float32),
            jax.ShapeDtypeStruct((_N, _H), jnp.float32),
        ),
    )(agg1, h1, dinv, W2, b1)


def _tc3_body(agg_ref, h2_ref, dinv_ref, op_ref, wp_ref, bp_ref, wfc_ref,
              bfc_ref, b2_ref, out_ref):
    dinv = dinv_ref[...]
    a = agg_ref[...]
    z = dinv * (a[0] + a[1]) + (dinv * dinv) * h2_ref[...] + b2_ref[...]
    r = jnp.maximum(z, 0.0)
    emb = jnp.dot(r, wp_ref[...], preferred_element_type=jnp.float32) \
        + bp_ref[...]
    wfc = wfc_ref[...]
    out = jnp.dot(emb, wfc[:128], preferred_element_type=jnp.float32) \
        + jnp.dot(op_ref[...], wfc[128:], preferred_element_type=jnp.float32) \
        + bfc_ref[...]
    out_ref[...] = out


def _tc3(agg2, h2, dinv, op, Wp, bp, Wfc, bfc, b2):
    return pl.pallas_call(
        _tc3_body,
        out_shape=jax.ShapeDtypeStruct((_N, 1), jnp.float32),
    )(agg2, h2, dinv, op, Wp, bp, Wfc, bfc, b2)


def kernel(x, edge_index, edge_attr, op, W1, b1, W2, b2, Wp, bp, Wfc, bfc):
    src = edge_index[0].reshape(_NMG, _MGRP, _CHUNK)
    dst = edge_index[1].reshape(_NMG, _MGRP, _CHUNK)
    attr = edge_attr[:, 0].reshape(_NCHUNKS, _CHUNK)

    ew, deg_parts = _sc_deg(attr, dst.reshape(_NCHUNKS, _CHUNK))
    ewi = jax.lax.bitcast_convert_type(ew.reshape(_NMG, _MGRP, _CHUNK),
                                       jnp.int32)
    pk = jnp.concatenate([src, dst, ewi], axis=1)

    h1 = _tca(x, W1)
    hp1, dinv = _tcb(h1, deg_parts)
    agg1 = _sc_msg(hp1, pk)
    h2, hp2 = _tc2(agg1, h1, dinv, W2, b1.reshape(1, _H))
    agg2 = _sc_msg(hp2, pk)
    return _tc3(agg2, h2, dinv, op, Wp, bp.reshape(1, 128),
                Wfc, bfc.reshape(1, 1), b2.reshape(1, _H))


# speedup vs baseline: 28.2031x; 1.0081x over previous
import functools

import jax
import jax.numpy as jnp
from jax import lax
from jax.experimental import pallas as pl
from jax.experimental.pallas import tpu as pltpu
from jax.experimental.pallas import tpu_sc as plsc

_N = 10000
_E = 320000
_H = 64
_CHUNK = 128
_NCHUNKS = _E // _CHUNK
_GRP = 4
_NGRP = _NCHUNKS // _GRP
_MGRP = 2
_NMG = _NCHUNKS // _MGRP
_NSLOT = 4
_NC = 2
_NS = 16
_NW = _NC * _NS
_BASE = _NCHUNKS // _NW
_REM = _NCHUNKS % _NW
_GBASE = _NGRP // _NW
_GREM = _NGRP % _NW
_MBASE = _NMG // _NW
_MREM = _NMG % _NW
_RPS = _N // _NS


def _mesh():
    return plsc.VectorSubcoreMesh(core_axis_name="c", subcore_axis_name="s")


def _worker_id():
    return lax.axis_index("s") * _NC + lax.axis_index("c")


def _chunk_range(w):
    start = w * _BASE + jnp.minimum(w, _REM)
    count = _BASE + (w < _REM).astype(jnp.int32)
    return start, start + count


def _group_range(w):
    start = w * _GBASE + jnp.minimum(w, _GREM)
    count = _GBASE + (w < _GREM).astype(jnp.int32)
    return start, start + count


def _mgroup_range(w):
    start = w * _MBASE + jnp.minimum(w, _MREM)
    count = _MBASE + (w < _MREM).astype(jnp.int32)
    return start, start + count


def _sc_deg_body(attr_hbm, dst_hbm, ew_hbm, deg_hbm, dst_v, attr_v, ew_v,
                 deg_local):
    w = _worker_id()

    @pl.loop(0, _N // 16)
    def _zero(i):
        deg_local[pl.ds(i * 16, 16)] = jnp.zeros((16,), jnp.float32)

    lo, hi = _group_range(w)

    @pl.loop(lo, hi)
    def _chunk(t):
        g4 = t * _GRP
        pltpu.sync_copy(dst_hbm.at[pl.ds(g4, _GRP)], dst_v)
        pltpu.sync_copy(attr_hbm.at[pl.ds(g4, _GRP)], attr_v)

        for j in range(_GRP):

            @pl.loop(0, _CHUNK // 16)
            def _grp(i):
                d16 = dst_v[j, pl.ds(i * 16, 16)]
                a16 = attr_v[j, pl.ds(i * 16, 16)]
                e16 = 1.0 / (a16 + 1.0)
                ew_v[j, pl.ds(i * 16, 16)] = e16
                plsc.addupdate_scatter(deg_local, [d16], e16)

        pltpu.sync_copy(ew_v, ew_hbm.at[pl.ds(g4, _GRP)])

    pltpu.sync_copy(deg_local, deg_hbm.at[w])


def _sc_deg(attr2, dst2):
    kern = functools.partial(
        pl.kernel,
        compiler_params=pltpu.CompilerParams(needs_layout_passes=False, use_tc_tiling_on_sc=False),
        out_type=(
            jax.ShapeDtypeStruct((_NCHUNKS, _CHUNK), jnp.float32),
            jax.ShapeDtypeStruct((_NW, _N), jnp.float32),
        ),
        mesh=_mesh(),
        scratch_types=[
            pltpu.VMEM((_GRP, _CHUNK), jnp.int32),
            pltpu.VMEM((_GRP, _CHUNK), jnp.float32),
            pltpu.VMEM((_GRP, _CHUNK), jnp.float32),
            pltpu.VMEM((_N,), jnp.float32),
        ],
    )(_sc_deg_body)
    return kern(attr2, dst2)


def _sc_msg_body(hp_hbm, pk_hbm, out_hbm, sd_v, rows_v, acc_sh, gsems,
                 ssems, esems):
    c = lax.axis_index("c")
    s = lax.axis_index("s")
    w = s * _NC + c

    @pl.loop(0, _CHUNK)
    def _zrow(i):
        for j in range(_H // 16):
            rows_v[0, 0, i, pl.ds(j * 16, 16)] = jnp.zeros((16,), jnp.float32)

    for k in range(_RPS // 125):
        pltpu.sync_copy(rows_v.at[0, 0, pl.ds(0, 125)],
                        acc_sh.at[pl.ds(s * _RPS + k * 125, 125)])
    plsc.subcore_barrier()

    lo, hi = _mgroup_range(w)

    def gather_cps(b):
        return [pltpu.make_async_copy(hp_hbm.at[sd_v.at[b, j]],
                                      rows_v.at[b, j], gsems[b])
                for j in range(_MGRP)]

    def scatter_cps(b):
        return [pltpu.make_async_copy(rows_v.at[b, j],
                                      acc_sh.at[sd_v.at[b, _MGRP + j]],
                                      ssems[b])
                for j in range(_MGRP)]

    def start_edges(t, b):
        pltpu.make_async_copy(pk_hbm.at[t], sd_v.at[b], esems[b]).start()

    def start_gather(b):
        pltpu.make_async_copy(pk_hbm.at[0], sd_v.at[b], esems[b]).wait()
        for cp in gather_cps(b):
            cp.start()

    def drain_scatter(b):
        for cp in scatter_cps(b):
            cp.wait()

    def scale(b):
        for j in range(_MGRP):
            rj = rows_v.at[b, j]
            ej = sd_v.at[b, 2 * _MGRP + j]

            @pl.loop(0, _CHUNK // 16)
            def _g16(g):
                ew16 = plsc.bitcast(ej[pl.ds(g * 16, 16)], jnp.float32)
                base = g * 16
                for r16 in range(16):
                    ridx = jnp.full((16,), r16, jnp.int32)
                    splat = ew16.at[ridx].get(mode="promise_in_bounds")
                    r = base + r16
                    for k in range(_H // 16):
                        v = rj[r, pl.ds(k * 16, 16)]
                        rj[r, pl.ds(k * 16, 16)] = v * splat

    def process(t, b):
        for cp in gather_cps(b):
            cp.wait()
        scale(b)
        for j in range(_MGRP):
            pltpu.async_copy(rows_v.at[b, j],
                             acc_sh.at[sd_v.at[b, _MGRP + j]],
                             ssems[b], add=True)

    for m in range(_NSLOT - 1):
        start_edges(lo + m, m)
    for m in range(_NSLOT - 2):
        start_gather(m)

    ntrip = (hi - lo + _NSLOT - 1) // _NSLOT

    @pl.loop(0, ntrip)
    def _trip(i):
        t0 = lo + _NSLOT * i
        for k in range(_NSLOT):
            tk = t0 + k
            pg = (k + _NSLOT - 2) % _NSLOT
            pe = (k + _NSLOT - 1) % _NSLOT

            @pl.when(tk < hi)
            def _sub():
                process(tk, k)

                @pl.when(tk + _NSLOT - 2 < hi)
                def _preg():
                    start_gather(pg)

                @pl.when(tk + _NSLOT - 1 < hi)
                def _pree():
                    if k == 0:
                        @pl.when(i > 0)
                        def _dr():
                            drain_scatter(pe)
                    else:
                        drain_scatter(pe)
                    start_edges(tk + _NSLOT - 1, pe)

    for b in range(_NSLOT):
        drain_scatter(b)

    plsc.subcore_barrier()
    pltpu.sync_copy(acc_sh.at[pl.ds(s * _RPS, _RPS)],
                    out_hbm.at[c, pl.ds(s * _RPS, _RPS)])


def _sc_msg(hp, pk):
    kern = functools.partial(
        pl.kernel,
        compiler_params=pltpu.CompilerParams(needs_layout_passes=False, use_tc_tiling_on_sc=False),
        out_type=jax.ShapeDtypeStruct((_NC, _N, _H), jnp.float32),
        mesh=_mesh(),
        scratch_types=[
            pltpu.VMEM((_NSLOT, 3 * _MGRP, _CHUNK), jnp.int32),
            pltpu.VMEM((_NSLOT, _MGRP, _CHUNK, _H), jnp.float32),
            pltpu.VMEM_SHARED((_N, _H), jnp.float32),
            [pltpu.SemaphoreType.DMA] * _NSLOT,
            [pltpu.SemaphoreType.DMA] * _NSLOT,
            [pltpu.SemaphoreType.DMA] * _NSLOT,
        ],
    )(_sc_msg_body)
    return kern(hp, pk)


_BLK = 2000
_NBLK = _N // _BLK


def _rowspec(d):
    return pl.BlockSpec((_BLK, d), lambda i: (i, 0))


def _fullspec(shape):
    return pl.BlockSpec(shape, lambda i: tuple(0 for _ in shape))


def _tca_body(x_ref, w1_ref, h1_ref):
    h1_ref[...] = jnp.dot(x_ref[...], w1_ref[...],
                          preferred_element_type=jnp.float32)


def _tca(x, W1):
    return pl.pallas_call(
        _tca_body,
        grid=(_NBLK,),
        in_specs=[_rowspec(128), _fullspec((128, _H))],
        out_specs=_rowspec(_H),
        out_shape=jax.ShapeDtypeStruct((_N, _H), jnp.float32),
    )(x, W1)


def _tcb_body(h1_ref, degp_ref, hp1_ref, dinv_ref):
    deg = jnp.sum(degp_ref[...], axis=0)[:, None] + 1.0
    dinv = jnp.where(deg > 0, lax.rsqrt(jnp.maximum(deg, 1e-12)), 0.0)
    hp1_ref[...] = h1_ref[...] * dinv
    dinv_ref[...] = dinv


def _tcb(h1, deg_parts):
    return pl.pallas_call(
        _tcb_body,
        out_shape=(
            jax.ShapeDtypeStruct((_N, _H), jnp.float32),
            jax.ShapeDtypeStruct((_N, 1), jnp.float32),
        ),
    )(h1, deg_parts)


def _tc2_body(agg_ref, h1_ref, dinv_ref, w2_ref, b1_ref, h2_ref, hp2_ref):
    dinv = dinv_ref[...]
    a = agg_ref[...]
    z = dinv * (a[0] + a[1]) + (dinv * dinv) * h1_ref[...] + b1_ref[...]
    r = jnp.maximum(z, 0.0)
    h2 = jnp.dot(r, w2_ref[...], preferred_element_type=jnp.float32)
    h2_ref[...] = h2
    hp2_ref[...] = h2 * dinv


def _tc2(agg1, h1, dinv, W2, b1):
    return pl.pallas_call(
        _tc2_body,
        grid=(_NBLK,),
        in_specs=[
            pl.BlockSpec((_NC, _BLK, _H), lambda i: (0, i, 0)),
            _rowspec(_H),
            _rowspec(1),
            _fullspec((_H, _H)),
            _fullspec((1, _H)),
        ],
        out_specs=(_rowspec(_H), _rowspec(_H)),
        out_shape=(
            jax.ShapeDtypeStruct((_N, _H), jnp.float32),
            jax.ShapeDtypeStruct((_N, _H), jnp.float32),
        ),
    )(agg1, h1, dinv, W2, b1)


def _tc3_body(agg_ref, h2_ref, dinv_ref, op_ref, wp_ref, bp_ref, wfc_ref,
              bfc_ref, b2_ref, out_ref):
    dinv = dinv_ref[...]
    a = agg_ref[...]
    z = dinv * (a[0] + a[1]) + (dinv * dinv) * h2_ref[...] + b2_ref[...]
    r = jnp.maximum(z, 0.0)
    emb = jnp.dot(r, wp_ref[...], preferred_element_type=jnp.float32) \
        + bp_ref[...]
    wfc = wfc_ref[...]
    out = jnp.dot(emb, wfc[:128], preferred_element_type=jnp.float32) \
        + jnp.dot(op_ref[...], wfc[128:], preferred_element_type=jnp.float32) \
        + bfc_ref[...]
    out_ref[...] = out


def _tc3(agg2, h2, dinv, op, Wp, bp, Wfc, bfc, b2):
    return pl.pallas_call(
        _tc3_body,
        grid=(_NBLK,),
        in_specs=[
            pl.BlockSpec((_NC, _BLK, _H), lambda i: (0, i, 0)),
            _rowspec(_H),
            _rowspec(1),
            _rowspec(32),
            _fullspec((_H, 128)),
            _fullspec((1, 128)),
            _fullspec((160, 1)),
            _fullspec((1, 1)),
            _fullspec((1, _H)),
        ],
        out_specs=_rowspec(1),
        out_shape=jax.ShapeDtypeStruct((_N, 1), jnp.float32),
    )(agg2, h2, dinv, op, Wp, bp, Wfc, bfc, b2)


def kernel(x, edge_index, edge_attr, op, W1, b1, W2, b2, Wp, bp, Wfc, bfc):
    src = edge_index[0].reshape(_NMG, _MGRP, _CHUNK)
    dst = edge_index[1].reshape(_NMG, _MGRP, _CHUNK)
    attr = edge_attr[:, 0].reshape(_NCHUNKS, _CHUNK)

    ew, deg_parts = _sc_deg(attr, dst.reshape(_NCHUNKS, _CHUNK))
    ewi = jax.lax.bitcast_convert_type(ew.reshape(_NMG, _MGRP, _CHUNK),
                                       jnp.int32)
    pk = jnp.concatenate([src, dst, ewi], axis=1)

    h1 = _tca(x, W1)
    hp1, dinv = _tcb(h1, deg_parts)
    agg1 = _sc_msg(hp1, pk)
    h2, hp2 = _tc2(agg1, h1, dinv, W2, b1.reshape(1, _H))
    agg2 = _sc_msg(hp2, pk)
    return _tc3(agg2, h2, dinv, op, Wp, bp.reshape(1, 128),
                Wfc, bfc.reshape(1, 1), b2.reshape(1, _H))
